# Initial kernel scaffold; baseline (speedup 1.0000x reference)
import jax
import jax.numpy as jnp
from jax.experimental import pallas as pl


def kernel(hidden_states, gate_w, new_gate_w, expert_gate_w, expert_up_w, expert_down_w):
    raise NotImplementedError("write your pallas kernel here")


# speedup vs baseline: 1.4757x; 1.4757x over previous
import jax
import jax.numpy as jnp
from jax.experimental import pallas as pl
from jax.experimental.pallas import tpu as pltpu

T, D = 2048, 1024
NE, NK, FF = 8, 2, 512
TAU = 0.5
BLK = 256
NTB = T // BLK


def _moe_body(x_ref, gw_ref, ngw_ref, gum_ref, eg_ref, eu_ref, ed_ref, out_ref):
    e = pl.program_id(1)
    xb = x_ref[...]
    xh = xb.astype(jnp.bfloat16)

    cdims = (((1,), (1,)), ((), ()))
    ol = jax.lax.dot_general(xh, gw_ref[...], cdims,
                             preferred_element_type=jnp.float32)
    rl = jax.lax.dot_general(xh, ngw_ref[...], cdims,
                             preferred_element_type=jnp.float32)
    rwts = jax.nn.softmax(ol, axis=-1)
    gl = (rl + gum_ref[...]) * (1.0 / TAU)

    ids = jax.lax.broadcasted_iota(jnp.int32, (BLK, NE), 1)
    m1 = jnp.max(gl, axis=1, keepdims=True)
    i1 = jnp.min(jnp.where(gl == m1, ids, NE), axis=1, keepdims=True)
    gl2 = jnp.where(ids == i1, -1e30, gl)
    m2 = jnp.max(gl2, axis=1, keepdims=True)
    i2 = jnp.min(jnp.where(gl2 == m2, ids, NE), axis=1, keepdims=True)
    w1 = jnp.sum(jnp.where(ids == i1, rwts, 0.0), axis=1, keepdims=True)
    w2 = jnp.sum(jnp.where(ids == i2, rwts, 0.0), axis=1, keepdims=True)
    we = jnp.where(i1 == e, w1, 0.0) + jnp.where(i2 == e, w2, 0.0)

    wg = eg_ref[0]
    wu = eu_ref[0]
    wd = ed_ref[0]
    g = jax.lax.dot_general(xh, wg, cdims, preferred_element_type=jnp.float32)
    u = jax.lax.dot_general(xh, wu, cdims, preferred_element_type=jnp.float32)
    h = (g * jax.nn.sigmoid(g) * u).astype(jnp.bfloat16)
    o = jax.lax.dot_general(h, wd, cdims, preferred_element_type=jnp.float32)

    @pl.when(e == 0)
    def _():
        out_ref[...] = jnp.zeros_like(out_ref)

    out_ref[...] += o * we


def kernel(hidden_states, gate_w, new_gate_w, expert_gate_w, expert_up_w,
           expert_down_w):
    x2d = hidden_states.reshape(T, D)
    gum = jax.random.gumbel(jax.random.key(42), (T, NE), dtype=jnp.float32)
    gwh = gate_w.astype(jnp.bfloat16)
    ngwh = new_gate_w.astype(jnp.bfloat16)
    egh = expert_gate_w.astype(jnp.bfloat16)
    euh = expert_up_w.astype(jnp.bfloat16)
    edh = expert_down_w.astype(jnp.bfloat16)

    out = pl.pallas_call(
        _moe_body,
        grid=(NTB, NE),
        in_specs=[
            pl.BlockSpec((BLK, D), lambda tb, e: (tb, 0)),
            pl.BlockSpec((NE, D), lambda tb, e: (0, 0)),
            pl.BlockSpec((NE, D), lambda tb, e: (0, 0)),
            pl.BlockSpec((BLK, NE), lambda tb, e: (tb, 0)),
            pl.BlockSpec((1, FF, D), lambda tb, e: (e, 0, 0)),
            pl.BlockSpec((1, FF, D), lambda tb, e: (e, 0, 0)),
            pl.BlockSpec((1, D, FF), lambda tb, e: (e, 0, 0)),
        ],
        out_specs=pl.BlockSpec((BLK, D), lambda tb, e: (tb, 0)),
        out_shape=jax.ShapeDtypeStruct((T, D), jnp.float32),
    )(x2d, gwh, ngwh, gum, egh, euh, edh)
    return out.reshape(hidden_states.shape)
